# Initial kernel scaffold; baseline (speedup 1.0000x reference)
import jax
import jax.numpy as jnp
from jax.experimental import pallas as pl


def kernel(h_V, h_E, edge_idx, W1_w, W1_b, W2_w, W2_b, W3_w, W3_b, D1_w, D1_b, D2_w, D2_b, ln1_g, ln1_b, ln2_g, ln2_b):
    raise NotImplementedError("write your pallas kernel here")


# speedup vs baseline: 1.7241x; 1.7241x over previous
import functools

import jax
import jax.numpy as jnp
from jax import lax
from jax.experimental import pallas as pl
from jax.experimental.pallas import tpu as pltpu
from jax.experimental.pallas import tpu_sc as plsc

N_NODES = 10000
N_EDGES = 320000
D = 128
DIN = 144
DFF = 512
SCALE = 30.0
EPS = 1e-5


EDGE_BLOCK = 2560


def _edge_mlp_body(he_ref, w1_ref, b1_ref, w2_ref, b2_ref, w3_ref, b3_ref,
                   out_ref):
    ct = (((1,), (1,)), ((), ()))
    h = he_ref[...]
    h = jax.lax.dot_general(h, w1_ref[...], ct,
                            preferred_element_type=jnp.float32)
    h = jnp.maximum(h + b1_ref[...], 0.0)
    h = jax.lax.dot_general(h, w2_ref[...], ct,
                            preferred_element_type=jnp.float32)
    h = jnp.maximum(h + b2_ref[...], 0.0)
    h = jax.lax.dot_general(h, w3_ref[...], ct,
                            preferred_element_type=jnp.float32)
    out_ref[...] = h + b3_ref[...]


def _edge_mlp(h_E, W1_w, W1_b, W2_w, W2_b, W3_w, W3_b):
    grid = N_EDGES // EDGE_BLOCK
    full = lambda i: (0, 0)
    return pl.pallas_call(
        _edge_mlp_body,
        grid=(grid,),
        in_specs=[
            pl.BlockSpec((EDGE_BLOCK, DIN), lambda i: (i, 0)),
            pl.BlockSpec((D, DIN), full),
            pl.BlockSpec((1, D), full),
            pl.BlockSpec((D, D), full),
            pl.BlockSpec((1, D), full),
            pl.BlockSpec((D, D), full),
            pl.BlockSpec((1, D), full),
        ],
        out_specs=pl.BlockSpec((EDGE_BLOCK, D), lambda i: (i, 0)),
        out_shape=jax.ShapeDtypeStruct((N_EDGES, D), jnp.float32),
    )(h_E, W1_w, W1_b.reshape(1, D), W2_w, W2_b.reshape(1, D), W3_w,
      W3_b.reshape(1, D))


NC = 2
NS = 16
NW = NC * NS
EPW = N_EDGES // NW
CHUNK = 80
NCHUNK = EPW // CHUNK
N_PAD = 10240
ZROWS = 128
ROWS_PER_TILE = N_PAD // NS


def _sc_scatter_body(msg_hbm, src_hbm, out_hbm, idx_v, msg_v, zero_v, acc):
    c = lax.axis_index("c")
    s = lax.axis_index("s")
    wid = c * NS + s

    def zstore(k, carry):
        i = k // (D // 16)
        j = k % (D // 16)
        zero_v[i, pl.ds(j * 16, 16)] = jnp.zeros((16,), jnp.float32)
        return carry
    lax.fori_loop(0, ZROWS * (D // 16), zstore, 0)
    for r in range(ROWS_PER_TILE // ZROWS):
        pltpu.sync_copy(zero_v, acc.at[pl.ds(s * ROWS_PER_TILE + r * ZROWS,
                                             ZROWS)])
    plsc.subcore_barrier()

    ebase = wid * EPW

    def chunk_step(j, carry):
        off = ebase + j * CHUNK
        pltpu.sync_copy(src_hbm.at[pl.ds(off, CHUNK)], idx_v)
        pltpu.sync_copy(msg_hbm.at[pl.ds(off, CHUNK)], msg_v)
        pltpu.sync_copy(msg_v, acc.at[idx_v], add=True)
        return carry
    lax.fori_loop(0, NCHUNK, chunk_step, 0)
    plsc.subcore_barrier()

    for r in range(ROWS_PER_TILE // ZROWS):
        rows = pl.ds(s * ROWS_PER_TILE + r * ZROWS, ZROWS)
        pltpu.sync_copy(acc.at[rows], out_hbm.at[c, rows])


def _sc_scatter(msg, src):
    mesh = plsc.VectorSubcoreMesh(core_axis_name="c", subcore_axis_name="s",
                                  num_cores=NC, num_subcores=NS)
    fn = pl.kernel(
        _sc_scatter_body,
        out_type=jax.ShapeDtypeStruct((NC, N_PAD, D), jnp.float32),
        mesh=mesh,
        scratch_types=[
            pltpu.VMEM((CHUNK,), jnp.int32),
            pltpu.VMEM((CHUNK, D), jnp.float32),
            pltpu.VMEM((ZROWS, D), jnp.float32),
            pltpu.VMEM_SHARED((N_PAD, D), jnp.float32),
        ],
    )
    return fn(msg, src)


NODE_BLOCK = 2000


def _layer_norm_in_kernel(x, g, b):
    mu = jnp.mean(x, axis=-1, keepdims=True)
    xc = x - mu
    var = jnp.mean(xc * xc, axis=-1, keepdims=True)
    return xc * jax.lax.rsqrt(var + EPS) * g + b


def _node_body(p_ref, hv_ref, d1_ref, b1_ref, d2_ref, b2_ref, g1_ref, bb1_ref,
               g2_ref, bb2_ref, out_ref):
    ct = (((1,), (1,)), ((), ()))
    dh = (p_ref[0] + p_ref[1]) * (1.0 / SCALE)
    x = _layer_norm_in_kernel(hv_ref[...] + dh, g1_ref[...], bb1_ref[...])
    y = jax.lax.dot_general(x, d1_ref[...], ct,
                            preferred_element_type=jnp.float32)
    y = jnp.maximum(y + b1_ref[...], 0.0)
    y = jax.lax.dot_general(y, d2_ref[...], ct,
                            preferred_element_type=jnp.float32)
    x = x + y + b2_ref[...]
    out_ref[...] = _layer_norm_in_kernel(x, g2_ref[...], bb2_ref[...])


def _node_update(partials, h_V, D1_w, D1_b, D2_w, D2_b, ln1_g, ln1_b, ln2_g,
                 ln2_b):
    grid = N_NODES // NODE_BLOCK
    full = lambda i: (0, 0)
    return pl.pallas_call(
        _node_body,
        grid=(grid,),
        in_specs=[
            pl.BlockSpec((NC, NODE_BLOCK, D), lambda i: (0, i, 0)),
            pl.BlockSpec((NODE_BLOCK, D), lambda i: (i, 0)),
            pl.BlockSpec((DFF, D), full),
            pl.BlockSpec((1, DFF), full),
            pl.BlockSpec((D, DFF), full),
            pl.BlockSpec((1, D), full),
            pl.BlockSpec((1, D), full),
            pl.BlockSpec((1, D), full),
            pl.BlockSpec((1, D), full),
            pl.BlockSpec((1, D), full),
        ],
        out_specs=pl.BlockSpec((NODE_BLOCK, D), lambda i: (i, 0)),
        out_shape=jax.ShapeDtypeStruct((N_NODES, D), jnp.float32),
    )(partials, h_V, D1_w, D1_b.reshape(1, DFF), D2_w, D2_b.reshape(1, D),
      ln1_g.reshape(1, D), ln1_b.reshape(1, D), ln2_g.reshape(1, D),
      ln2_b.reshape(1, D))


def kernel(h_V, h_E, edge_idx, W1_w, W1_b, W2_w, W2_b, W3_w, W3_b, D1_w, D1_b,
           D2_w, D2_b, ln1_g, ln1_b, ln2_g, ln2_b):
    msg = _edge_mlp(h_E, W1_w, W1_b, W2_w, W2_b, W3_w, W3_b)
    partials = _sc_scatter(msg, edge_idx[0])
    return _node_update(partials, h_V, D1_w, D1_b, D2_w, D2_b, ln1_g, ln1_b,
                        ln2_g, ln2_b)


# speedup vs baseline: 2.1253x; 1.2327x over previous
import functools

import jax
import jax.numpy as jnp
from jax import lax
from jax.experimental import pallas as pl
from jax.experimental.pallas import tpu as pltpu
from jax.experimental.pallas import tpu_sc as plsc

N_NODES = 10000
N_EDGES = 320000
D = 128
DIN = 144
DFF = 512
SCALE = 30.0
EPS = 1e-5


EDGE_BLOCK = 2560


def _edge_mlp_body(he_ref, w1_ref, b1_ref, w2_ref, b2_ref, w3_ref, b3_ref,
                   out_ref):
    ct = (((1,), (1,)), ((), ()))
    h = he_ref[...].astype(jnp.bfloat16)
    h = jax.lax.dot_general(h, w1_ref[...], ct,
                            preferred_element_type=jnp.float32)
    h = jnp.maximum(h + b1_ref[...], 0.0).astype(jnp.bfloat16)
    h = jax.lax.dot_general(h, w2_ref[...], ct,
                            preferred_element_type=jnp.float32)
    h = jnp.maximum(h + b2_ref[...], 0.0).astype(jnp.bfloat16)
    h = jax.lax.dot_general(h, w3_ref[...], ct,
                            preferred_element_type=jnp.float32)
    out_ref[...] = h + b3_ref[...]


def _edge_mlp(h_E, W1_w, W1_b, W2_w, W2_b, W3_w, W3_b):
    grid = N_EDGES // EDGE_BLOCK
    full = lambda i: (0, 0)
    return pl.pallas_call(
        _edge_mlp_body,
        grid=(grid,),
        in_specs=[
            pl.BlockSpec((EDGE_BLOCK, DIN), lambda i: (i, 0)),
            pl.BlockSpec((D, DIN), full),
            pl.BlockSpec((1, D), full),
            pl.BlockSpec((D, D), full),
            pl.BlockSpec((1, D), full),
            pl.BlockSpec((D, D), full),
            pl.BlockSpec((1, D), full),
        ],
        out_specs=pl.BlockSpec((EDGE_BLOCK, D), lambda i: (i, 0)),
        out_shape=jax.ShapeDtypeStruct((N_EDGES, D), jnp.float32),
    )(h_E, W1_w.astype(jnp.bfloat16), W1_b.reshape(1, D),
      W2_w.astype(jnp.bfloat16), W2_b.reshape(1, D),
      W3_w.astype(jnp.bfloat16), W3_b.reshape(1, D))


NC = 2
NS = 16
NW = NC * NS
EPW = N_EDGES // NW
CHUNK = 40
NCHUNK = EPW // CHUNK
N_PAD = 10240
ZROWS = 80
ROWS_PER_TILE = N_PAD // NS


NBUF = 5
NGROUP = NCHUNK // NBUF


def _sc_scatter_body(msg_hbm, src_hbm, out_hbm, idx_v, msg_v, zero_v, acc,
                     cp_sems, sc_sems):
    c = lax.axis_index("c")
    s = lax.axis_index("s")
    wid = c * NS + s
    ebase = wid * EPW

    def start_copies(b, off):
        pltpu.async_copy(src_hbm.at[pl.ds(off, CHUNK)], idx_v.at[b],
                         cp_sems.at[b])
        pltpu.async_copy(msg_hbm.at[pl.ds(off, CHUNK)], msg_v.at[b],
                         cp_sems.at[b])

    def wait_copies(b):
        pltpu.make_async_copy(src_hbm.at[pl.ds(0, CHUNK)], idx_v.at[b],
                              cp_sems.at[b]).wait()
        pltpu.make_async_copy(msg_hbm.at[pl.ds(0, CHUNK)], msg_v.at[b],
                              cp_sems.at[b]).wait()

    def start_scatter(b):
        pltpu.async_copy(msg_v.at[b], acc.at[idx_v.at[b]], sc_sems.at[b],
                         add=True)

    def wait_scatter(b):
        pltpu.make_async_copy(msg_v.at[b], acc.at[idx_v.at[b]],
                              sc_sems.at[b]).wait()

    for b in range(NBUF):
        start_copies(b, ebase + b * CHUNK)

    def zstore(k, carry):
        i = k // (D // 16)
        j = k % (D // 16)
        zero_v[i, pl.ds(j * 16, 16)] = jnp.zeros((16,), jnp.float32)
        return carry
    lax.fori_loop(0, ZROWS * (D // 16), zstore, 0)
    for r in range(ROWS_PER_TILE // ZROWS):
        pltpu.sync_copy(zero_v, acc.at[pl.ds(s * ROWS_PER_TILE + r * ZROWS,
                                             ZROWS)])
    plsc.subcore_barrier()

    def group_step(g, carry):
        for b in range(NBUF):
            wait_copies(b)
            start_scatter(b)
        for b in range(NBUF):
            wait_scatter(b)
            start_copies(b, ebase + ((g + 1) * NBUF + b) * CHUNK)
        return carry
    lax.fori_loop(0, NGROUP - 1, group_step, 0)
    for b in range(NBUF):
        wait_copies(b)
        start_scatter(b)
    for b in range(NBUF):
        wait_scatter(b)
    plsc.subcore_barrier()

    for r in range(ROWS_PER_TILE // ZROWS):
        rows = pl.ds(s * ROWS_PER_TILE + r * ZROWS, ZROWS)
        pltpu.sync_copy(acc.at[rows], out_hbm.at[c, rows])


def _sc_scatter(msg, src):
    mesh = plsc.VectorSubcoreMesh(core_axis_name="c", subcore_axis_name="s",
                                  num_cores=NC, num_subcores=NS)
    fn = pl.kernel(
        _sc_scatter_body,
        out_type=jax.ShapeDtypeStruct((NC, N_PAD, D), jnp.float32),
        mesh=mesh,
        scratch_types=[
            pltpu.VMEM((NBUF, CHUNK), jnp.int32),
            pltpu.VMEM((NBUF, CHUNK, D), jnp.float32),
            pltpu.VMEM((ZROWS, D), jnp.float32),
            pltpu.VMEM_SHARED((N_PAD, D), jnp.float32),
            pltpu.SemaphoreType.DMA((NBUF,)),
            pltpu.SemaphoreType.DMA((NBUF,)),
        ],
    )
    return fn(msg, src)


NODE_BLOCK = 2000


def _layer_norm_in_kernel(x, g, b):
    mu = jnp.mean(x, axis=-1, keepdims=True)
    xc = x - mu
    var = jnp.mean(xc * xc, axis=-1, keepdims=True)
    return xc * jax.lax.rsqrt(var + EPS) * g + b


def _node_body(p_ref, hv_ref, d1_ref, b1_ref, d2_ref, b2_ref, g1_ref, bb1_ref,
               g2_ref, bb2_ref, out_ref):
    ct = (((1,), (1,)), ((), ()))
    dh = (p_ref[0] + p_ref[1]) * (1.0 / SCALE)
    x = _layer_norm_in_kernel(hv_ref[...] + dh, g1_ref[...], bb1_ref[...])
    y = jax.lax.dot_general(x, d1_ref[...], ct,
                            preferred_element_type=jnp.float32)
    y = jnp.maximum(y + b1_ref[...], 0.0)
    y = jax.lax.dot_general(y, d2_ref[...], ct,
                            preferred_element_type=jnp.float32)
    x = x + y + b2_ref[...]
    out_ref[...] = _layer_norm_in_kernel(x, g2_ref[...], bb2_ref[...])


def _node_update(partials, h_V, D1_w, D1_b, D2_w, D2_b, ln1_g, ln1_b, ln2_g,
                 ln2_b):
    grid = N_NODES // NODE_BLOCK
    full = lambda i: (0, 0)
    return pl.pallas_call(
        _node_body,
        grid=(grid,),
        in_specs=[
            pl.BlockSpec((NC, NODE_BLOCK, D), lambda i: (0, i, 0)),
            pl.BlockSpec((NODE_BLOCK, D), lambda i: (i, 0)),
            pl.BlockSpec((DFF, D), full),
            pl.BlockSpec((1, DFF), full),
            pl.BlockSpec((D, DFF), full),
            pl.BlockSpec((1, D), full),
            pl.BlockSpec((1, D), full),
            pl.BlockSpec((1, D), full),
            pl.BlockSpec((1, D), full),
            pl.BlockSpec((1, D), full),
        ],
        out_specs=pl.BlockSpec((NODE_BLOCK, D), lambda i: (i, 0)),
        out_shape=jax.ShapeDtypeStruct((N_NODES, D), jnp.float32),
    )(partials, h_V, D1_w, D1_b.reshape(1, DFF), D2_w, D2_b.reshape(1, D),
      ln1_g.reshape(1, D), ln1_b.reshape(1, D), ln2_g.reshape(1, D),
      ln2_b.reshape(1, D))


def kernel(h_V, h_E, edge_idx, W1_w, W1_b, W2_w, W2_b, W3_w, W3_b, D1_w, D1_b,
           D2_w, D2_b, ln1_g, ln1_b, ln2_g, ln2_b):
    msg = _edge_mlp(h_E, W1_w, W1_b, W2_w, W2_b, W3_w, W3_b)
    partials = _sc_scatter(msg, edge_idx[0])
    return _node_update(partials, h_V, D1_w, D1_b, D2_w, D2_b, ln1_g, ln1_b,
                        ln2_g, ln2_b)


# speedup vs baseline: 4.1359x; 1.9461x over previous
import functools

import jax
import jax.numpy as jnp
from jax import lax
from jax.experimental import pallas as pl
from jax.experimental.pallas import tpu as pltpu
from jax.experimental.pallas import tpu_sc as plsc

N_NODES = 10000
N_EDGES = 320000
D = 128
DIN = 144
DFF = 512
SCALE = 30.0
EPS = 1e-5


EDGE_BLOCK = 2560


def _edge_mlp_body(he_ref, w1_ref, b1_ref, w2_ref, b2_ref, w3_ref, b3_ref,
                   out_ref):
    ct = (((1,), (1,)), ((), ()))
    h = he_ref[...].astype(jnp.bfloat16)
    h = jax.lax.dot_general(h, w1_ref[...], (((0,), (1,)), ((), ())),
                            preferred_element_type=jnp.float32)
    h = jnp.maximum(h + b1_ref[...], 0.0).astype(jnp.bfloat16)
    h = jax.lax.dot_general(h, w2_ref[...], ct,
                            preferred_element_type=jnp.float32)
    h = jnp.maximum(h + b2_ref[...], 0.0).astype(jnp.bfloat16)
    h = jax.lax.dot_general(h, w3_ref[...], ct,
                            preferred_element_type=jnp.float32)
    out_ref[...] = h + b3_ref[...]


def _edge_mlp(h_E, W1_w, W1_b, W2_w, W2_b, W3_w, W3_b):
    grid = N_EDGES // EDGE_BLOCK
    full = lambda i: (0, 0)
    return pl.pallas_call(
        _edge_mlp_body,
        grid=(grid,),
        in_specs=[
            pl.BlockSpec((DIN, EDGE_BLOCK), lambda i: (0, i)),
            pl.BlockSpec((D, DIN), full),
            pl.BlockSpec((1, D), full),
            pl.BlockSpec((D, D), full),
            pl.BlockSpec((1, D), full),
            pl.BlockSpec((D, D), full),
            pl.BlockSpec((1, D), full),
        ],
        out_specs=pl.BlockSpec((EDGE_BLOCK, D), lambda i: (i, 0)),
        out_shape=jax.ShapeDtypeStruct((N_EDGES, D), jnp.float32),
    )(h_E.T, W1_w.astype(jnp.bfloat16), W1_b.reshape(1, D),
      W2_w.astype(jnp.bfloat16), W2_b.reshape(1, D),
      W3_w.astype(jnp.bfloat16), W3_b.reshape(1, D))


NC = 2
NS = 16
NW = NC * NS
EPW = N_EDGES // NW
CHUNK = 40
NCHUNK = EPW // CHUNK
N_PAD = 10240
ZROWS = 80
ROWS_PER_TILE = N_PAD // NS


NBUF = 5
NGROUP = NCHUNK // NBUF


def _sc_scatter_body(msg_hbm, src_hbm, out_hbm, idx_v, msg_v, zero_v, acc,
                     cp_sems, sc_sems):
    c = lax.axis_index("c")
    s = lax.axis_index("s")
    wid = c * NS + s
    ebase = wid * EPW

    def start_copies(b, off):
        pltpu.async_copy(src_hbm.at[pl.ds(off, CHUNK)], idx_v.at[b],
                         cp_sems.at[b])
        pltpu.async_copy(msg_hbm.at[pl.ds(off, CHUNK)], msg_v.at[b],
                         cp_sems.at[b])

    def wait_copies(b):
        pltpu.make_async_copy(src_hbm.at[pl.ds(0, CHUNK)], idx_v.at[b],
                              cp_sems.at[b]).wait()
        pltpu.make_async_copy(msg_hbm.at[pl.ds(0, CHUNK)], msg_v.at[b],
                              cp_sems.at[b]).wait()

    def start_scatter(b):
        pltpu.async_copy(msg_v.at[b], acc.at[idx_v.at[b]], sc_sems.at[b],
                         add=True)

    def wait_scatter(b):
        pltpu.make_async_copy(msg_v.at[b], acc.at[idx_v.at[b]],
                              sc_sems.at[b]).wait()

    for b in range(NBUF):
        start_copies(b, ebase + b * CHUNK)

    def zstore(k, carry):
        i = k // (D // 16)
        j = k % (D // 16)
        zero_v[i, pl.ds(j * 16, 16)] = jnp.zeros((16,), jnp.float32)
        return carry
    lax.fori_loop(0, ZROWS * (D // 16), zstore, 0)
    for r in range(ROWS_PER_TILE // ZROWS):
        pltpu.sync_copy(zero_v, acc.at[pl.ds(s * ROWS_PER_TILE + r * ZROWS,
                                             ZROWS)])
    plsc.subcore_barrier()

    def group_step(g, carry):
        for b in range(NBUF):
            wait_copies(b)
            start_scatter(b)
        for b in range(NBUF):
            wait_scatter(b)
            start_copies(b, ebase + ((g + 1) * NBUF + b) * CHUNK)
        return carry
    lax.fori_loop(0, NGROUP - 1, group_step, 0)
    for b in range(NBUF):
        wait_copies(b)
        start_scatter(b)
    for b in range(NBUF):
        wait_scatter(b)
    plsc.subcore_barrier()

    for r in range(ROWS_PER_TILE // ZROWS):
        rows = pl.ds(s * ROWS_PER_TILE + r * ZROWS, ZROWS)
        pltpu.sync_copy(acc.at[rows], out_hbm.at[c, rows])


def _sc_scatter(msg, src):
    mesh = plsc.VectorSubcoreMesh(core_axis_name="c", subcore_axis_name="s",
                                  num_cores=NC, num_subcores=NS)
    fn = pl.kernel(
        _sc_scatter_body,
        out_type=jax.ShapeDtypeStruct((NC, N_PAD, D), jnp.float32),
        mesh=mesh,
        scratch_types=[
            pltpu.VMEM((NBUF, CHUNK), jnp.int32),
            pltpu.VMEM((NBUF, CHUNK, D), jnp.float32),
            pltpu.VMEM((ZROWS, D), jnp.float32),
            pltpu.VMEM_SHARED((N_PAD, D), jnp.float32),
            pltpu.SemaphoreType.DMA((NBUF,)),
            pltpu.SemaphoreType.DMA((NBUF,)),
        ],
    )
    return fn(msg, src)


NODE_BLOCK = 2000


def _layer_norm_in_kernel(x, g, b):
    mu = jnp.mean(x, axis=-1, keepdims=True)
    xc = x - mu
    var = jnp.mean(xc * xc, axis=-1, keepdims=True)
    return xc * jax.lax.rsqrt(var + EPS) * g + b


def _node_body(p_ref, hv_ref, d1_ref, b1_ref, d2_ref, b2_ref, g1_ref, bb1_ref,
               g2_ref, bb2_ref, out_ref):
    ct = (((1,), (1,)), ((), ()))
    dh = (p_ref[0] + p_ref[1]) * (1.0 / SCALE)
    x = _layer_norm_in_kernel(hv_ref[...] + dh, g1_ref[...], bb1_ref[...])
    y = jax.lax.dot_general(x, d1_ref[...], ct,
                            preferred_element_type=jnp.float32)
    y = jnp.maximum(y + b1_ref[...], 0.0)
    y = jax.lax.dot_general(y, d2_ref[...], ct,
                            preferred_element_type=jnp.float32)
    x = x + y + b2_ref[...]
    out_ref[...] = _layer_norm_in_kernel(x, g2_ref[...], bb2_ref[...])


def _node_update(partials, h_V, D1_w, D1_b, D2_w, D2_b, ln1_g, ln1_b, ln2_g,
                 ln2_b):
    grid = N_NODES // NODE_BLOCK
    full = lambda i: (0, 0)
    return pl.pallas_call(
        _node_body,
        grid=(grid,),
        in_specs=[
            pl.BlockSpec((NC, NODE_BLOCK, D), lambda i: (0, i, 0)),
            pl.BlockSpec((NODE_BLOCK, D), lambda i: (i, 0)),
            pl.BlockSpec((DFF, D), full),
            pl.BlockSpec((1, DFF), full),
            pl.BlockSpec((D, DFF), full),
            pl.BlockSpec((1, D), full),
            pl.BlockSpec((1, D), full),
            pl.BlockSpec((1, D), full),
            pl.BlockSpec((1, D), full),
            pl.BlockSpec((1, D), full),
        ],
        out_specs=pl.BlockSpec((NODE_BLOCK, D), lambda i: (i, 0)),
        out_shape=jax.ShapeDtypeStruct((N_NODES, D), jnp.float32),
    )(partials, h_V, D1_w, D1_b.reshape(1, DFF), D2_w, D2_b.reshape(1, D),
      ln1_g.reshape(1, D), ln1_b.reshape(1, D), ln2_g.reshape(1, D),
      ln2_b.reshape(1, D))


def kernel(h_V, h_E, edge_idx, W1_w, W1_b, W2_w, W2_b, W3_w, W3_b, D1_w, D1_b,
           D2_w, D2_b, ln1_g, ln1_b, ln2_g, ln2_b):
    msg = _edge_mlp(h_E, W1_w, W1_b, W2_w, W2_b, W3_w, W3_b)
    partials = _sc_scatter(msg, edge_idx[0])
    return _node_update(partials, h_V, D1_w, D1_b, D2_w, D2_b, ln1_g, ln1_b,
                        ln2_g, ln2_b)
